# Initial kernel scaffold; baseline (speedup 1.0000x reference)
import jax
import jax.numpy as jnp
from jax.experimental import pallas as pl


def kernel(x, pos, batch, W_c1a, b_c1a, W_c1b, b_c1b, W_c2a, b_c2a, W_c2b, b_c2b, W_c3a, b_c3a, W_c3b, b_c3b, W_l1, b_l1, W_m1, b_m1, W_m2, b_m2, W_m3, b_m3):
    raise NotImplementedError("write your pallas kernel here")


# speedup vs baseline: 5.1287x; 5.1287x over previous
import functools

import jax
import jax.numpy as jnp
from jax.experimental import pallas as pl
from jax.experimental.pallas import tpu as pltpu

_B, _P, _K = 16, 1024, 30
_HI = jax.lax.Precision.HIGHEST


def _sortable_keys(d):
    bits = jax.lax.bitcast_convert_type(d, jnp.int32)
    return bits ^ ((bits >> 31) & jnp.int32(0x7FFFFFFF))


def _gather_rows(oh_b, tab_hi, tab_lo):
    g = jax.lax.dot_general(oh_b, tab_hi, (((1,), (0,)), ((), ())),
                            preferred_element_type=jnp.float32)
    g += jax.lax.dot_general(oh_b, tab_lo, (((1,), (0,)), ((), ())),
                             preferred_element_type=jnp.float32)
    return g


def _edge_conv(x, W1, b1, W2, b2, dist_precision):
    d = x.shape[1]
    d2 = jnp.sum(x * x, axis=1)
    G = jax.lax.dot_general(x, x, (((1,), (1,)), ((), ())),
                            precision=dist_precision)
    D = d2[:, None] + d2[None, :] - 2.0 * G
    keys0 = _sortable_keys(D)

    W1a, W1b = W1[:d], W1[d:]
    A = jax.lax.dot_general(x, W1a - W1b, (((1,), (0,)), ((), ())),
                            precision=_HI) + b1
    Bm = jax.lax.dot_general(x, W1b, (((1,), (0,)), ((), ())), precision=_HI)
    tab_hi = Bm.astype(jnp.bfloat16)
    tab_lo = (Bm - tab_hi.astype(jnp.float32)).astype(jnp.bfloat16)

    iota = jax.lax.broadcasted_iota(jnp.int32, (_P, _P), 1)

    def body(_, carry):
        keys, acc = carry
        m = jnp.min(keys, axis=1)
        sel = keys == m[:, None]
        idx = jnp.min(jnp.where(sel, iota, jnp.int32(_P)), axis=1)
        onehot = iota == idx[:, None]
        keys = jnp.where(onehot, jnp.int32(2147483647), keys)
        g = _gather_rows(onehot.astype(jnp.bfloat16), tab_hi, tab_lo)
        h1 = jnp.maximum(A + g, 0.0)
        h2 = jax.lax.dot_general(h1, W2, (((1,), (0,)), ((), ())),
                                 precision=_HI) + b2
        acc = jnp.maximum(acc, jnp.maximum(h2, 0.0))
        return keys, acc

    acc0 = jnp.full((_P, 64), -jnp.inf, jnp.float32)
    _, acc = jax.lax.fori_loop(0, _K, body, (keys0, acc0))
    return acc


def _mm(a, w, b):
    return jax.lax.dot_general(a, w, (((1,), (0,)), ((), ())),
                               precision=_HI) + b


def _net_kernel(x0_ref, Wc1a, bc1a, Wc1b, bc1b, Wc2a, bc2a, Wc2b, bc2b,
                Wc3a, bc3a, Wc3b, bc3b, Wl1, bl1, Wm1, bm1, Wm2, bm2,
                Wm3, bm3, out_ref, dist_precision):
    x0 = x0_ref[0]
    x1 = _edge_conv(x0, Wc1a[...], bc1a[...], Wc1b[...], bc1b[...],
                    dist_precision)
    x2 = _edge_conv(x1, Wc2a[...], bc2a[...], Wc2b[...], bc2b[...],
                    dist_precision)
    x3 = _edge_conv(x2, Wc3a[...], bc3a[...], Wc3b[...], bc3b[...],
                    dist_precision)
    feat = jnp.concatenate([x1, x2, x3], axis=1)
    h = jnp.maximum(_mm(feat, Wl1[...], bl1[...]), 0.0)
    h = jnp.maximum(_mm(h, Wm1[...], bm1[...]), 0.0)
    h = jnp.maximum(_mm(h, Wm2[...], bm2[...]), 0.0)
    h = _mm(h, Wm3[...], bm3[...])
    m = jnp.max(h, axis=1, keepdims=True)
    s = h - m
    lse = jnp.log(jnp.sum(jnp.exp(s), axis=1, keepdims=True))
    out_ref[0] = s - lse


def kernel(x, pos, batch, W_c1a, b_c1a, W_c1b, b_c1b, W_c2a, b_c2a,
           W_c2b, b_c2b, W_c3a, b_c3a, W_c3b, b_c3b, W_l1, b_l1,
           W_m1, b_m1, W_m2, b_m2, W_m3, b_m3):
    x0 = jnp.concatenate([x, pos], axis=1).reshape(_B, _P, 9)
    ws = [W_c1a, b_c1a, W_c1b, b_c1b, W_c2a, b_c2a, W_c2b, b_c2b,
          W_c3a, b_c3a, W_c3b, b_c3b, W_l1, b_l1, W_m1, b_m1,
          W_m2, b_m2, W_m3, b_m3]

    def w_spec(a):
        nd = a.ndim
        return pl.BlockSpec(a.shape, lambda b, _n=nd: (0,) * _n)

    out = pl.pallas_call(
        functools.partial(_net_kernel, dist_precision=_HI),
        grid=(_B,),
        in_specs=[pl.BlockSpec((1, _P, 9), lambda b: (b, 0, 0))]
        + [w_spec(a) for a in ws],
        out_specs=pl.BlockSpec((1, _P, 40), lambda b: (b, 0, 0)),
        out_shape=jax.ShapeDtypeStruct((_B, _P, 40), jnp.float32),
    )(x0, *ws)
    return out.reshape(_B * _P, 40)
